# baseline (speedup 1.0000x reference)
import functools

import jax
import jax.numpy as jnp
from jax import lax
from jax.experimental import pallas as pl
from jax.experimental.pallas import tpu as pltpu
from jax.experimental.pallas import tpu_sc as plsc

_NC = 2
_NS = 16
_NW = _NC * _NS
_CB = 256
_L = 16


def _build(B0, B1, V, D):
    bw = B0 // _NW
    mesh = plsc.VectorSubcoreMesh(core_axis_name="c", subcore_axis_name="s")

    @functools.partial(
        pl.kernel,
        mesh=mesh,
        out_type=jax.ShapeDtypeStruct((B1, D, B0), jnp.float32),
        compiler_params=pltpu.CompilerParams(needs_layout_passes=False),
        scratch_types=[
            pltpu.VMEM((_CB,), jnp.int32),
            pltpu.VMEM((_CB,), jnp.int32),
            pltpu.VMEM((_CB, 128), jnp.float32),
            pltpu.VMEM((_CB, 128), jnp.float32),
            pltpu.VMEM((D, _CB), jnp.float32),
            pltpu.VMEM((D, _CB), jnp.float32),
            pltpu.SemaphoreType.DMA,
            pltpu.SemaphoreType.DMA,
            pltpu.SemaphoreType.DMA,
            pltpu.SemaphoreType.DMA,
        ],
    )
    def k(idxT_hbm, tp_hbm, outT_hbm, ia, ib, ga, gb, oa, ob, gsa, gsb, wsa, wsb):
        wid = lax.axis_index("s") * _NC + lax.axis_index("c")
        b0 = wid * bw
        lanes = lax.iota(jnp.int32, _L)

        def gather(s, c, idx_v, gbuf, gsem):
            pltpu.sync_copy(idxT_hbm.at[s, pl.ds(b0 + c * _CB, _CB)], idx_v)
            return pltpu.async_copy(tp_hbm.at[idx_v], gbuf, gsem)

        def emit(s, c, gbuf, obuf, wsem):
            for e in range(D):
                evec = jnp.full((_L,), e, jnp.int32)
                for j in range(_CB // _L):
                    obuf[e, pl.ds(j * _L, _L)] = plsc.load_gather(
                        gbuf, [lanes + (j * _L), evec]
                    )
            return pltpu.async_copy(
                obuf,
                outT_hbm.at[s, pl.ds(0, D), pl.ds(b0 + c * _CB, _CB)],
                wsem,
            )

        gh_a = gather(0, 0, ia, ga, gsa)
        gh_b = gather(0, 1, ib, gb, gsb)
        gh_a.wait()
        wh_a = emit(0, 0, ga, oa, wsa)
        gather(1, 0, ia, ga, gsa)
        gh_b.wait()
        wh_b = emit(0, 1, gb, ob, wsb)
        gather(1, 1, ib, gb, gsb)

        def body(s, carry):
            gh_a.wait()
            wh_a.wait()
            emit(s, 0, ga, oa, wsa)
            gather(s + 1, 0, ia, ga, gsa)
            gh_b.wait()
            wh_b.wait()
            emit(s, 1, gb, ob, wsb)
            gather(s + 1, 1, ib, gb, gsb)
            return carry

        lax.fori_loop(1, B1 - 1, body, 0)
        gh_a.wait()
        wh_a.wait()
        emit(B1 - 1, 0, ga, oa, wsa)
        gh_b.wait()
        wh_b.wait()
        emit(B1 - 1, 1, gb, ob, wsb)
        wh_a.wait()
        wh_b.wait()

    return k


def kernel(input, table):
    B0, B1 = input.shape
    V, D = table.shape
    tp = jnp.pad(table, ((0, 0), (0, 128 - D)))
    outT = _build(B0, B1, V, D)(input.T.astype(jnp.int32), tp)
    return outT.transpose(2, 0, 1)
